# Initial kernel scaffold; baseline (speedup 1.0000x reference)
import functools

import jax
import jax.numpy as jnp
from jax import lax
from jax.experimental import pallas as pl
from jax.experimental.pallas import tpu as pltpu
from jax.experimental.pallas import tpu_sc as plsc

_NUM_CORES = 2
_NUM_SUBCORES = 16
_NUM_WORKERS = _NUM_CORES * _NUM_SUBCORES
_CHUNK = 128


@functools.lru_cache(maxsize=None)
def _make_gather(B: int, D: int):
    assert B % (_NUM_WORKERS * _CHUNK) == 0
    b_per_w = B // _NUM_WORKERS
    n_chunks = b_per_w // _CHUNK
    mesh = plsc.VectorSubcoreMesh(core_axis_name="c", subcore_axis_name="s")

    @functools.partial(
        pl.kernel,
        mesh=mesh,
        out_type=jax.ShapeDtypeStruct((B, D), jnp.float32),
        scratch_types=[
            pltpu.VMEM((b_per_w,), jnp.int32),
            pltpu.VMEM((_CHUNK, D), jnp.float32),
            pltpu.SemaphoreType.DMA,
        ],
    )
    def gather_kernel(idx_hbm, table_hbm, out_hbm, idx_v, rows_v, sem):
        wid = lax.axis_index("s") * _NUM_CORES + lax.axis_index("c")
        base = wid * b_per_w
        pltpu.sync_copy(idx_hbm.at[pl.ds(base, b_per_w)], idx_v)
        for i in range(n_chunks):
            pltpu.async_copy(
                table_hbm.at[idx_v.at[pl.ds(i * _CHUNK, _CHUNK)]], rows_v, sem
            ).wait()
            pltpu.sync_copy(rows_v, out_hbm.at[pl.ds(base + i * _CHUNK, _CHUNK)])

    return gather_kernel


def kernel(input, table):
    s0, s1 = input.shape
    d = table.shape[1]
    idx = input.reshape(-1).astype(jnp.int32)
    out = _make_gather(idx.shape[0], d)(idx, table)
    return out.reshape(s0, s1, d)


# speedup vs baseline: 1.4387x; 1.4387x over previous
import functools

import jax
import jax.numpy as jnp
from jax import lax
from jax.experimental import pallas as pl
from jax.experimental.pallas import tpu as pltpu
from jax.experimental.pallas import tpu_sc as plsc

_NC = 2
_NS = 16
_NW = _NC * _NS
_SUP = 512
_L = 16


def _splat(x, dtype=jnp.int32):
    return jnp.full((_L,), 0, dtype) + x


@functools.lru_cache(maxsize=None)
def _make_gather(B: int, D: int, V: int):
    n_full = (V // _SUP)
    n_sup = n_full + 1
    n_pairs = (n_sup + 2 * _NW - 1) // (2 * _NW)
    mesh = plsc.VectorSubcoreMesh(core_axis_name="c", subcore_axis_name="s")

    @functools.partial(
        pl.kernel,
        mesh=mesh,
        out_type=jax.ShapeDtypeStruct((B + _NW, 128), jnp.float32),
        scratch_types=[
            pltpu.VMEM((B,), jnp.int32),
            pltpu.VMEM((B,), jnp.int32),
            pltpu.VMEM((D, _SUP), jnp.float32),
            pltpu.VMEM((D, _SUP), jnp.float32),
            pltpu.VMEM((32,), jnp.int32),
            pltpu.VMEM((32,), jnp.int32),
            pltpu.VMEM((128, 128), jnp.float32),
            pltpu.VMEM((1, 128), jnp.int32),
            pltpu.SMEM((8,), jnp.int32),
            pltpu.SemaphoreType.DMA,
            pltpu.SemaphoreType.DMA,
        ],
        compiler_params=pltpu.CompilerParams(
            use_tc_tiling_on_sc=True, needs_layout_passes=False
        ),
    )
    def gather_kernel(idx_hbm, table_hbm, tail_hbm, out_hbm, idx_v, blist,
                      buf0, buf1, pend_r, pend_b, rows_v, bchunk, sc,
                      sem, sem2):
        wid = lax.axis_index("s") * _NC + lax.axis_index("c")
        trash = B + wid
        iota = lax.iota(jnp.int32, _L)

        pltpu.sync_copy(idx_hbm, idx_v)
        sc[0] = 0
        sc[1] = 0
        sc[2] = 0
        for t8 in range(8):
            bchunk[0, pl.ds(t8 * _L, _L)] = _splat(trash)

        def scan_body(i, carry):
            v = idx_v[pl.ds(i * _L, _L)]
            j = lax.shift_right_logical(v, 9)
            m = lax.bitwise_and(j, _splat(_NW - 1)) == _splat(wid)
            cnt = sc[2]
            pos = cnt + plsc.cumsum(jnp.where(m, 1, 0).astype(jnp.int32)) - 1
            plsc.store_scatter(blist, [pos], iota + i * _L, mask=m)
            sc[2] = cnt + plsc.all_reduce_population_count(m)[0]
            return carry

        lax.fori_loop(0, B // _L, scan_body, 0)

        def flush():
            pltpu.async_copy(rows_v, out_hbm.at[bchunk.at[0]], sem2).wait()
            for t8 in range(8):
                bchunk[0, pl.ds(t8 * _L, _L)] = _splat(trash)
            sc[0] = 0

        def maybe_flush():
            @pl.when(sc[0] > 128 - _L)
            def _():
                flush()

        def extract(buf, r16, b16, m, nh):
            maybe_flush()
            w0 = sc[0]
            col16 = lax.bitwise_and(r16, _splat(_SUP - 1))
            slot16 = w0 + iota
            for d in range(D):
                vals = plsc.load_gather(buf, [_splat(d), col16], mask=m)
                plsc.store_scatter(rows_v, [slot16, _splat(d)], vals, mask=m)
            plsc.store_scatter(bchunk, [_splat(0), slot16], b16, mask=m)
            sc[0] = w0 + nh

        def process(s, buf):
            cnt = sc[2]
            ngrp = lax.shift_right_logical(cnt + (_L - 1), 4)

            def grp(i, carry):
                valid = (iota + i * _L) < _splat(cnt)
                b16 = blist[pl.ds(i * _L, _L)]
                b16 = jnp.where(valid, b16, 0)
                r16 = plsc.load_gather(idx_v, [b16])
                m = jnp.logical_and(
                    valid, lax.shift_right_logical(r16, 9) == _splat(s)
                )
                nh = plsc.all_reduce_population_count(m)[0]

                @pl.when(nh > 0)
                def _():
                    pc = sc[1]
                    ppos = (
                        pc
                        + plsc.cumsum(jnp.where(m, 1, 0).astype(jnp.int32))
                        - 1
                    )
                    plsc.store_scatter(pend_r, [ppos], r16, mask=m)
                    plsc.store_scatter(pend_b, [ppos], b16, mask=m)
                    sc[1] = pc + nh

                    @pl.when(sc[1] >= _L)
                    def _():
                        pr = pend_r[pl.ds(0, _L)]
                        pb = pend_b[pl.ds(0, _L)]
                        full = _splat(1) > _splat(0)
                        extract(buf, pr, pb, full, _L)
                        pend_r[pl.ds(0, _L)] = pend_r[pl.ds(_L, _L)]
                        pend_b[pl.ds(0, _L)] = pend_b[pl.ds(_L, _L)]
                        sc[1] = sc[1] - _L

                return carry

            lax.fori_loop(0, ngrp, grp, 0)

            @pl.when(sc[1] > 0)
            def _():
                pc = sc[1]
                pr = pend_r[pl.ds(0, _L)]
                pb = pend_b[pl.ds(0, _L)]
                mv = iota < _splat(pc)
                extract(buf, pr, pb, mv, pc)
                sc[1] = 0

        def fetch(s, buf):
            @pl.when(s < n_full)
            def _():
                pltpu.async_copy(
                    table_hbm.at[:, pl.ds(s * _SUP, _SUP)], buf, sem
                )

            @pl.when(s == n_full)
            def _():
                pltpu.async_copy(tail_hbm, buf, sem)

        def drain(buf):
            pltpu.make_async_copy(
                table_hbm.at[:, pl.ds(0, _SUP)], buf, sem
            ).wait()

        fetch(wid, buf0)

        def pair(p, carry):
            s0 = wid + (2 * p) * _NW
            s1 = s0 + _NW
            s2 = s0 + 2 * _NW

            @pl.when(s1 < n_sup)
            def _():
                fetch(s1, buf1)

            @pl.when(s0 < n_sup)
            def _():
                drain(buf0)
                process(s0, buf0)

            @pl.when(s2 < n_sup)
            def _():
                fetch(s2, buf0)

            @pl.when(s1 < n_sup)
            def _():
                drain(buf1)
                process(s1, buf1)

            return carry

        lax.fori_loop(0, n_pairs, pair, 0)
        flush()

    return gather_kernel


def kernel(input, table):
    s0, s1 = input.shape
    v, d = table.shape
    idx = input.reshape(-1).astype(jnp.int32)
    b = idx.shape[0]
    table_t = table.T
    n_full = v // _SUP
    tail = table_t[:, n_full * _SUP:]
    tail_pad = jnp.pad(tail, ((0, 0), (0, _SUP - tail.shape[1])))
    out_raw = _make_gather(b, d, v)(idx, table_t, tail_pad)
    return out_raw[:b, :d].reshape(s0, s1, d)


# speedup vs baseline: 1.4415x; 1.0019x over previous
import functools

import jax
import jax.numpy as jnp
from jax import lax
from jax.experimental import pallas as pl
from jax.experimental.pallas import tpu as pltpu
from jax.experimental.pallas import tpu_sc as plsc

_NC = 2
_NS = 16
_NW = _NC * _NS
_SUP = 512
_L = 16


def _splat(x, dtype=jnp.int32):
    return jnp.full((_L,), 0, dtype) + x


@functools.lru_cache(maxsize=None)
def _make_gather(B: int, D: int, V: int):
    n_full = (V // _SUP)
    n_sup = n_full + 1
    n_pairs = (n_sup + 2 * _NW - 1) // (2 * _NW)
    mesh = plsc.VectorSubcoreMesh(core_axis_name="c", subcore_axis_name="s")

    @functools.partial(
        pl.kernel,
        mesh=mesh,
        out_type=jax.ShapeDtypeStruct((B + _NW, 128), jnp.float32),
        scratch_types=[
            pltpu.VMEM((B,), jnp.int32),
            pltpu.VMEM((B,), jnp.int32),
            pltpu.VMEM((D, _SUP), jnp.float32),
            pltpu.VMEM((D, _SUP), jnp.float32),
            pltpu.VMEM((32,), jnp.int32),
            pltpu.VMEM((32,), jnp.int32),
            pltpu.VMEM((1024,), jnp.int32),
            pltpu.VMEM((128, 128), jnp.float32),
            pltpu.VMEM((1, 128), jnp.int32),
            pltpu.SMEM((8,), jnp.int32),
            pltpu.SemaphoreType.DMA,
            pltpu.SemaphoreType.DMA,
        ],
        compiler_params=pltpu.CompilerParams(
            use_tc_tiling_on_sc=True, needs_layout_passes=False
        ),
    )
    def gather_kernel(idx_hbm, table_hbm, tail_hbm, out_hbm, idx_v, blist,
                      buf0, buf1, pend_r, pend_b, occ, rows_v, bchunk, sc,
                      sem, sem2):
        wid = lax.axis_index("s") * _NC + lax.axis_index("c")
        trash = B + wid
        iota = lax.iota(jnp.int32, _L)

        pltpu.sync_copy(idx_hbm, idx_v)
        sc[0] = 0
        sc[1] = 0
        sc[2] = 0
        for t8 in range(8):
            bchunk[0, pl.ds(t8 * _L, _L)] = _splat(trash)
        zero = _splat(0)
        for z in range(1024 // _L):
            occ[pl.ds(z * _L, _L)] = zero

        def scan_body(i, carry):
            v = idx_v[pl.ds(i * _L, _L)]
            j = lax.shift_right_logical(v, 9)
            m = lax.bitwise_and(j, _splat(_NW - 1)) == _splat(wid)
            cnt = sc[2]
            pos = cnt + plsc.cumsum(jnp.where(m, 1, 0).astype(jnp.int32)) - 1
            plsc.store_scatter(blist, [pos], iota + i * _L, mask=m)
            slot = lax.shift_left(lax.shift_right_logical(j, 5), 4) + (
                lax.bitwise_and(lax.shift_right_logical(v, 7), _splat(3))
            )
            plsc.store_scatter(occ, [slot], _splat(1), mask=m)
            sc[2] = cnt + plsc.all_reduce_population_count(m)[0]
            return carry

        lax.fori_loop(0, B // _L, scan_body, 0)

        def flush():
            pltpu.async_copy(rows_v, out_hbm.at[bchunk.at[0]], sem2).wait()
            for t8 in range(8):
                bchunk[0, pl.ds(t8 * _L, _L)] = _splat(trash)
            sc[0] = 0

        def maybe_flush():
            @pl.when(sc[0] > 128 - _L)
            def _():
                flush()

        def extract(buf, r16, b16, m, nh):
            maybe_flush()
            w0 = sc[0]
            col16 = lax.bitwise_and(r16, _splat(_SUP - 1))
            slot16 = w0 + iota
            for d in range(D):
                vals = plsc.load_gather(buf, [_splat(d), col16], mask=m)
                plsc.store_scatter(rows_v, [slot16, _splat(d)], vals, mask=m)
            plsc.store_scatter(bchunk, [_splat(0), slot16], b16, mask=m)
            sc[0] = w0 + nh

        def process(s, buf):
            cnt = sc[2]
            ngrp = lax.shift_right_logical(cnt + (_L - 1), 4)

            def grp(i, carry):
                valid = (iota + i * _L) < _splat(cnt)
                b16 = blist[pl.ds(i * _L, _L)]
                b16 = jnp.where(valid, b16, 0)
                r16 = plsc.load_gather(idx_v, [b16])
                m = jnp.logical_and(
                    valid, lax.shift_right_logical(r16, 9) == _splat(s)
                )
                nh = plsc.all_reduce_population_count(m)[0]

                @pl.when(nh > 0)
                def _():
                    pc = sc[1]
                    ppos = (
                        pc
                        + plsc.cumsum(jnp.where(m, 1, 0).astype(jnp.int32))
                        - 1
                    )
                    plsc.store_scatter(pend_r, [ppos], r16, mask=m)
                    plsc.store_scatter(pend_b, [ppos], b16, mask=m)
                    sc[1] = pc + nh

                    @pl.when(sc[1] >= _L)
                    def _():
                        pr = pend_r[pl.ds(0, _L)]
                        pb = pend_b[pl.ds(0, _L)]
                        full = _splat(1) > _splat(0)
                        extract(buf, pr, pb, full, _L)
                        pend_r[pl.ds(0, _L)] = pend_r[pl.ds(_L, _L)]
                        pend_b[pl.ds(0, _L)] = pend_b[pl.ds(_L, _L)]
                        sc[1] = sc[1] - _L

                return carry

            lax.fori_loop(0, ngrp, grp, 0)

            @pl.when(sc[1] > 0)
            def _():
                pc = sc[1]
                pr = pend_r[pl.ds(0, _L)]
                pb = pend_b[pl.ds(0, _L)]
                mv = iota < _splat(pc)
                extract(buf, pr, pb, mv, pc)
                sc[1] = 0

        def sub_conds(s):
            t = lax.shift_right_logical(s, 5)
            flags = occ[pl.ds(t * _L, _L)] > zero
            return [
                plsc.all_reduce_population_count(
                    jnp.logical_and(flags, iota == _splat(k))
                )[0]
                > 0
                for k in range(4)
            ]

        def fetch(s, buf):
            conds = sub_conds(s)

            @pl.when(s < n_full)
            def _():
                for k in range(4):
                    @pl.when(conds[k])
                    def _(k=k):
                        pltpu.async_copy(
                            table_hbm.at[:, pl.ds(s * _SUP + k * 128, 128)],
                            buf.at[:, pl.ds(k * 128, 128)],
                            sem,
                        )

            @pl.when(s == n_full)
            def _():
                @pl.when(conds[0])
                def _():
                    pltpu.async_copy(
                        tail_hbm.at[:, pl.ds(0, 128)],
                        buf.at[:, pl.ds(0, 128)],
                        sem,
                    )

        def drain(s, buf):
            conds = sub_conds(s)
            for k in range(4):
                cond = conds[k]
                if k > 0:
                    cond = jnp.logical_and(cond, s < n_full)

                @pl.when(cond)
                def _(k=k):
                    pltpu.make_async_copy(
                        table_hbm.at[:, pl.ds(0, 128)],
                        buf.at[:, pl.ds(k * 128, 128)],
                        sem,
                    ).wait()

        fetch(wid, buf0)

        def pair(p, carry):
            s0 = wid + (2 * p) * _NW
            s1 = s0 + _NW
            s2 = s0 + 2 * _NW

            @pl.when(s1 < n_sup)
            def _():
                fetch(s1, buf1)

            @pl.when(s0 < n_sup)
            def _():
                drain(s0, buf0)
                process(s0, buf0)

            @pl.when(s2 < n_sup)
            def _():
                fetch(s2, buf0)

            @pl.when(s1 < n_sup)
            def _():
                drain(s1, buf1)
                process(s1, buf1)

            return carry

        lax.fori_loop(0, n_pairs, pair, 0)
        flush()

    return gather_kernel


def kernel(input, table):
    s0, s1 = input.shape
    v, d = table.shape
    idx = input.reshape(-1).astype(jnp.int32)
    b = idx.shape[0]
    table_t = table.T
    n_full = v // _SUP
    tail = table_t[:, n_full * _SUP:]
    tail_pad = jnp.pad(tail, ((0, 0), (0, _SUP - tail.shape[1])))
    out_raw = _make_gather(b, d, v)(idx, table_t, tail_pad)
    return out_raw[:b, :d].reshape(s0, s1, d)
